# baseline (speedup 1.0000x reference)
import jax
import jax.numpy as jnp
from jax.experimental import pallas as pl
from jax.experimental.pallas import tpu as pltpu

ROWS = 59049
BATCH = 128
SUB = 243
GROUP = 3 * SUB
BLK_GROUPS = 8
BLK = GROUP * BLK_GROUPS


def _interleave(re, im):
    return jnp.stack([re, im], axis=1).reshape(2 * SUB, BATCH)


def _rot_kernel(ang_ref, x_ref, y_ref):
    half = 0.5 * ang_ref[0]
    c = jnp.cos(half)
    ns = -jnp.sin(half)
    for g in range(BLK_GROUPS):
        b0 = g * GROUP
        x0 = x_ref[b0:b0 + SUB, :]
        x1 = x_ref[b0 + SUB:b0 + 2 * SUB, :]
        x2 = x_ref[b0 + 2 * SUB:b0 + 3 * SUB, :]
        o0 = 2 * b0
        y_ref[o0:o0 + 2 * SUB, :] = _interleave(c * x0, ns * x1)
        y_ref[o0 + 2 * SUB:o0 + 4 * SUB, :] = _interleave(c * x1, ns * x0)
        y_ref[o0 + 4 * SUB:o0 + 6 * SUB, :] = _interleave(
            x2, jnp.zeros_like(x2))


def kernel(x, angle):
    grid = (pl.cdiv(ROWS, BLK),)
    out = pl.pallas_call(
        _rot_kernel,
        grid=grid,
        in_specs=[
            pl.BlockSpec(memory_space=pltpu.SMEM),
            pl.BlockSpec((BLK, BATCH), lambda t: (t, 0)),
        ],
        out_specs=pl.BlockSpec((2 * BLK, BATCH), lambda t: (t, 0)),
        out_shape=jax.ShapeDtypeStruct((2 * ROWS, BATCH), jnp.float32),
    )(angle, x)
    z = out.reshape(ROWS, 2, BATCH)
    return jax.lax.complex(z[:, 0, :], z[:, 1, :])


# speedup vs baseline: 1.2863x; 1.2863x over previous
import functools

import jax
import jax.numpy as jnp
from jax import lax
from jax.experimental import pallas as pl
from jax.experimental.pallas import tpu as pltpu
from jax.experimental.pallas import tpu_sc as plsc

ROWS = 59049
BATCH = 128
SUB = 243
CH = 81
CHW = CH * BATCH
SUBW = SUB * BATCH
NCHUNKS = ROWS // CH
NC = 2
NS = 16
NW = NC * NS
LANES = 16

_MESH = plsc.VectorSubcoreMesh(core_axis_name="c", subcore_axis_name="s")


def _sc_body(x_hbm, cs_hbm, re_hbm, im_hbm, cs_v, in_v, re_v, im_v):
    wid = lax.axis_index("s") * NC + lax.axis_index("c")
    pltpu.sync_copy(cs_hbm, cs_v)
    cvec = cs_v[pl.ds(0, LANES)]
    nsvec = cs_v[pl.ds(LANES, LANES)]
    onev = cs_v[pl.ds(2 * LANES, LANES)]
    zerov = cs_v[pl.ds(3 * LANES, LANES)]

    n_k = (NCHUNKS - wid + NW - 1) // NW

    def chunk_body(i, _):
        k = wid + i * NW
        a = (k // 3) % 3
        w0 = k * CHW
        re_scale = jnp.where(a < 2, cvec, onev)
        im_scale = jnp.where(a < 2, nsvec, zerov)
        im_w0 = w0 + SUBW * jnp.where(a == 0, 1, 0) - SUBW * jnp.where(a == 1, 1, 0)

        pltpu.sync_copy(x_hbm.at[pl.ds(w0, CHW)], in_v)

        def row_body(r, _):
            base = r * BATCH
            for c in range(BATCH // LANES):
                o = base + c * LANES
                v = in_v[pl.ds(o, LANES)]
                re_v[pl.ds(o, LANES)] = v * re_scale
                im_v[pl.ds(o, LANES)] = v * im_scale
            return 0

        lax.fori_loop(0, CH, row_body, 0)
        pltpu.sync_copy(re_v, re_hbm.at[pl.ds(w0, CHW)])
        pltpu.sync_copy(im_v, im_hbm.at[pl.ds(im_w0, CHW)])
        return 0

    lax.fori_loop(0, n_k, chunk_body, 0)


def _sc_planes(x1, cs):
    run = pl.kernel(
        _sc_body,
        mesh=_MESH,
        out_type=[
            jax.ShapeDtypeStruct((ROWS * BATCH,), jnp.float32),
            jax.ShapeDtypeStruct((ROWS * BATCH,), jnp.float32),
        ],
        scratch_types=[
            pltpu.VMEM((4 * LANES,), jnp.float32),
            pltpu.VMEM((CHW,), jnp.float32),
            pltpu.VMEM((CHW,), jnp.float32),
            pltpu.VMEM((CHW,), jnp.float32),
        ],
    )
    return run(x1, cs)


def kernel(x, angle):
    half = 0.5 * angle[0]
    c = jnp.cos(half)
    ns = -jnp.sin(half)
    cs = jnp.concatenate([
        jnp.full((LANES,), c, jnp.float32),
        jnp.full((LANES,), ns, jnp.float32),
        jnp.ones((LANES,), jnp.float32),
        jnp.zeros((LANES,), jnp.float32),
    ])
    re, im = _sc_planes(x.reshape(ROWS * BATCH), cs)
    return jax.lax.complex(re.reshape(ROWS, BATCH), im.reshape(ROWS, BATCH))


# speedup vs baseline: 1.3509x; 1.0503x over previous
import jax
import jax.numpy as jnp
from jax import lax
from jax.experimental import pallas as pl
from jax.experimental.pallas import tpu as pltpu
from jax.experimental.pallas import tpu_sc as plsc

ROWS = 59049
BATCH = 128
SUB = 243
CH = 81
CHW = CH * BATCH
SUBW = SUB * BATCH
NCHUNKS = ROWS // CH
NC = 2
NS = 16
NW = NC * NS
KMAX = (NCHUNKS + NW - 1) // NW
LANES = 16

_MESH = plsc.VectorSubcoreMesh(core_axis_name="c", subcore_axis_name="s")


def _sc_body(x_hbm, cs_hbm, re_hbm, im_hbm, cs_v,
             in_v0, in_v1, re_v0, re_v1, im_v0, im_v1,
             sin0, sin1, sre0, sre1, sim0, sim1):
    in_v = (in_v0, in_v1)
    re_v = (re_v0, re_v1)
    im_v = (im_v0, im_v1)
    sin = (sin0, sin1)
    sre = (sre0, sre1)
    sim = (sim0, sim1)

    wid = lax.axis_index("s") * NC + lax.axis_index("c")
    pltpu.sync_copy(cs_hbm, cs_v)
    cvec = cs_v[pl.ds(0, LANES)]
    nsvec = cs_v[pl.ds(LANES, LANES)]
    onev = cs_v[pl.ds(2 * LANES, LANES)]
    zerov = cs_v[pl.ds(3 * LANES, LANES)]

    def chunk_info(j):
        k = wid + j * NW
        valid = k < NCHUNKS
        a = (k // 3) % 3
        w0 = k * CHW
        im_w0 = (w0 + SUBW * jnp.where(a == 0, 1, 0)
                 - SUBW * jnp.where(a == 1, 1, 0))
        return k, valid, a, w0, im_w0

    def start_in(j):
        _, valid, _, w0, _ = chunk_info(j)
        p = j % 2

        @pl.when(valid)
        def _():
            pltpu.make_async_copy(
                x_hbm.at[pl.ds(w0, CHW)], in_v[p], sin[p]).start()

    start_in(0)
    for j in range(KMAX):
        p = j % 2
        if j + 1 < KMAX:
            start_in(j + 1)
        _, valid, a, w0, im_w0 = chunk_info(j)

        @pl.when(valid)
        def _(j=j, p=p, a=a, w0=w0, im_w0=im_w0):
            pltpu.make_async_copy(
                x_hbm.at[pl.ds(w0, CHW)], in_v[p], sin[p]).wait()
            if j >= 2:
                pltpu.make_async_copy(
                    re_v[p], re_hbm.at[pl.ds(0, CHW)], sre[p]).wait()
                pltpu.make_async_copy(
                    im_v[p], im_hbm.at[pl.ds(0, CHW)], sim[p]).wait()
            re_scale = jnp.where(a < 2, cvec, onev)
            im_scale = jnp.where(a < 2, nsvec, zerov)

            def row_body(r, _):
                base = r * BATCH
                for c in range(BATCH // LANES):
                    o = base + c * LANES
                    v = in_v[p][pl.ds(o, LANES)]
                    re_v[p][pl.ds(o, LANES)] = v * re_scale
                    im_v[p][pl.ds(o, LANES)] = v * im_scale
                return 0

            lax.fori_loop(0, CH, row_body, 0)
            pltpu.make_async_copy(
                re_v[p], re_hbm.at[pl.ds(w0, CHW)], sre[p]).start()
            pltpu.make_async_copy(
                im_v[p], im_hbm.at[pl.ds(im_w0, CHW)], sim[p]).start()

    for j in (KMAX - 2, KMAX - 1):
        _, valid, _, _, _ = chunk_info(j)
        p = j % 2

        @pl.when(valid)
        def _(p=p):
            pltpu.make_async_copy(
                re_v[p], re_hbm.at[pl.ds(0, CHW)], sre[p]).wait()
            pltpu.make_async_copy(
                im_v[p], im_hbm.at[pl.ds(0, CHW)], sim[p]).wait()


def _sc_planes(x1, cs):
    run = pl.kernel(
        _sc_body,
        mesh=_MESH,
        out_type=[
            jax.ShapeDtypeStruct((ROWS * BATCH,), jnp.float32),
            jax.ShapeDtypeStruct((ROWS * BATCH,), jnp.float32),
        ],
        scratch_types=[
            pltpu.VMEM((4 * LANES,), jnp.float32),
            pltpu.VMEM((CHW,), jnp.float32),
            pltpu.VMEM((CHW,), jnp.float32),
            pltpu.VMEM((CHW,), jnp.float32),
            pltpu.VMEM((CHW,), jnp.float32),
            pltpu.VMEM((CHW,), jnp.float32),
            pltpu.VMEM((CHW,), jnp.float32),
            pltpu.SemaphoreType.DMA,
            pltpu.SemaphoreType.DMA,
            pltpu.SemaphoreType.DMA,
            pltpu.SemaphoreType.DMA,
            pltpu.SemaphoreType.DMA,
            pltpu.SemaphoreType.DMA,
        ],
    )
    return run(x1, cs)


def kernel(x, angle):
    half = 0.5 * angle[0]
    c = jnp.cos(half)
    ns = -jnp.sin(half)
    cs = jnp.concatenate([
        jnp.full((LANES,), c, jnp.float32),
        jnp.full((LANES,), ns, jnp.float32),
        jnp.ones((LANES,), jnp.float32),
        jnp.zeros((LANES,), jnp.float32),
    ])
    re, im = _sc_planes(x.reshape(ROWS * BATCH), cs)
    return jax.lax.complex(re.reshape(ROWS, BATCH), im.reshape(ROWS, BATCH))
